# baseline (speedup 1.0000x reference)
import jax
import jax.numpy as jnp
from jax.experimental import pallas as pl
from jax.experimental.pallas import tpu as pltpu

N = 1024
HID = 128
V = 2
F = HID // 2
BLOCK_D = 256
GRID = N // BLOCK_D

_NORMAL = (((1,), (0,)), ((), ()))
_T_DIMNUMS = (((0,), (0,)), ((), ()))


def _bignn_kernel(x_ref, afw_ref, abw_ref, wfw_ref, bfw_ref, wbw_ref,
                  bbw_ref, w1_ref, b1_ref, out_ref,
                  hfw_ref, hbw_ref, buf_fw, buf_bw, sem_fw, sem_bw):
    x = x_ref[...]
    for w_ref, b_ref, h_ref in ((wfw_ref, bfw_ref, hfw_ref),
                                (wbw_ref, bbw_ref, hbw_ref)):
        for i in range(V):
            h = (jnp.dot(x, w_ref[i], preferred_element_type=jnp.float32)
                 + b_ref[i:i + 1, :])
            h_ref[:, pl.ds(i * N, N)] = jnp.swapaxes(
                h.astype(jnp.bfloat16), 0, 1)

    def _copy(j, slot):
        d = pl.ds(j * BLOCK_D, BLOCK_D)
        pltpu.make_async_copy(afw_ref.at[:, :, d], buf_fw.at[slot],
                              sem_fw.at[slot]).start()
        pltpu.make_async_copy(abw_ref.at[:, :, d], buf_bw.at[slot],
                              sem_bw.at[slot]).start()

    _copy(0, 0)
    _copy(1, 1)
    for j in range(GRID):
        slot = j % 2
        pltpu.make_async_copy(afw_ref.at[:, :, pl.ds(j * BLOCK_D, BLOCK_D)],
                              buf_fw.at[slot], sem_fw.at[slot]).wait()
        pltpu.make_async_copy(abw_ref.at[:, :, pl.ds(j * BLOCK_D, BLOCK_D)],
                              buf_bw.at[slot], sem_bw.at[slot]).wait()

        parts = []
        for buf, h_ref in ((buf_bw, hbw_ref), (buf_fw, hfw_ref)):
            acc = None
            for i in range(V):
                a = buf[slot, i].astype(jnp.bfloat16)
                agg_t = jax.lax.dot_general(
                    h_ref[:, pl.ds(i * N, N)], a, _NORMAL,
                    preferred_element_type=jnp.float32)
                r = jnp.maximum(agg_t, 0.0)
                acc = r if acc is None else acc + r
            parts.append(acc)
        summed_t = jnp.concatenate(parts, axis=0)

        if j + 2 < GRID:
            _copy(j + 2, slot)

        d = pl.ds(j * BLOCK_D, BLOCK_D)
        feats = (jax.lax.dot_general(summed_t, w1_ref[...], _T_DIMNUMS,
                                     preferred_element_type=jnp.float32)
                 + b1_ref[...] + x_ref[d, :])
        out_ref[d, :] = feats


@jax.jit
def kernel(inps, fw_adjs, bw_adjs, W_fw, b_fw, W_bw, b_bw, W1, b1):
    out = pl.pallas_call(
        _bignn_kernel,
        in_specs=[
            pl.BlockSpec(memory_space=pltpu.MemorySpace.VMEM),
            pl.BlockSpec(memory_space=pl.ANY),
            pl.BlockSpec(memory_space=pl.ANY),
            pl.BlockSpec(memory_space=pltpu.MemorySpace.VMEM),
            pl.BlockSpec(memory_space=pltpu.MemorySpace.VMEM),
            pl.BlockSpec(memory_space=pltpu.MemorySpace.VMEM),
            pl.BlockSpec(memory_space=pltpu.MemorySpace.VMEM),
            pl.BlockSpec(memory_space=pltpu.MemorySpace.VMEM),
            pl.BlockSpec(memory_space=pltpu.MemorySpace.VMEM),
        ],
        out_specs=pl.BlockSpec(memory_space=pltpu.MemorySpace.VMEM),
        out_shape=jax.ShapeDtypeStruct((N, HID), jnp.float32),
        scratch_shapes=[
            pltpu.VMEM((F, V * N), jnp.bfloat16),
            pltpu.VMEM((F, V * N), jnp.bfloat16),
            pltpu.VMEM((2, V, N, BLOCK_D), jnp.int32),
            pltpu.VMEM((2, V, N, BLOCK_D), jnp.int32),
            pltpu.SemaphoreType.DMA((2,)),
            pltpu.SemaphoreType.DMA((2,)),
        ],
    )(inps, fw_adjs, bw_adjs, W_fw, b_fw, W_bw, b_bw, W1,
      b1.reshape(1, HID))
    return out


# speedup vs baseline: 1.0077x; 1.0077x over previous
import jax
import jax.numpy as jnp
from jax.experimental import pallas as pl
from jax.experimental.pallas import tpu as pltpu

N = 1024
HID = 128
V = 2
F = HID // 2
BLOCK_D = 256
GRID = N // BLOCK_D

_NORMAL = (((1,), (0,)), ((), ()))
_T_DIMNUMS = (((0,), (0,)), ((), ()))


def _bignn_kernel(x_ref, afw_ref, abw_ref, wfw_ref, bfw_ref, wbw_ref,
                  bbw_ref, w1_ref, b1_ref, out_ref,
                  hfw_ref, hbw_ref, buf_fw, buf_bw, sem_fw, sem_bw):
    x = x_ref[...]
    for w_ref, b_ref, h_ref in ((wfw_ref, bfw_ref, hfw_ref),
                                (wbw_ref, bbw_ref, hbw_ref)):
        for i in range(V):
            h = (jnp.dot(x, w_ref[i], preferred_element_type=jnp.float32)
                 + b_ref[i:i + 1, :])
            h_ref[:, pl.ds(i * N, N)] = jnp.swapaxes(
                h.astype(jnp.bfloat16), 0, 1)

    def _copies(j, slot):
        d = pl.ds(j * BLOCK_D, BLOCK_D)
        return [
            pltpu.make_async_copy(afw_ref.at[i, :, d], buf_fw.at[slot, i],
                                  sem_fw.at[slot, i])
            for i in range(V)
        ] + [
            pltpu.make_async_copy(abw_ref.at[i, :, d], buf_bw.at[slot, i],
                                  sem_bw.at[slot, i])
            for i in range(V)
        ]

    def _copy(j, slot):
        for c in _copies(j, slot):
            c.start()

    _copy(0, 0)
    _copy(1, 1)
    for j in range(GRID):
        slot = j % 2
        for c in _copies(j, slot):
            c.wait()

        parts = []
        for buf, h_ref in ((buf_bw, hbw_ref), (buf_fw, hfw_ref)):
            acc = None
            for i in range(V):
                a = buf[slot, i].astype(jnp.bfloat16)
                agg_t = jax.lax.dot_general(
                    h_ref[:, pl.ds(i * N, N)], a, _NORMAL,
                    preferred_element_type=jnp.float32)
                r = jnp.maximum(agg_t, 0.0)
                acc = r if acc is None else acc + r
            parts.append(acc)
        summed_t = jnp.concatenate(parts, axis=0)

        if j + 2 < GRID:
            _copy(j + 2, slot)

        d = pl.ds(j * BLOCK_D, BLOCK_D)
        feats = (jax.lax.dot_general(summed_t, w1_ref[...], _T_DIMNUMS,
                                     preferred_element_type=jnp.float32)
                 + b1_ref[...] + x_ref[d, :])
        out_ref[d, :] = feats


@jax.jit
def kernel(inps, fw_adjs, bw_adjs, W_fw, b_fw, W_bw, b_bw, W1, b1):
    out = pl.pallas_call(
        _bignn_kernel,
        in_specs=[
            pl.BlockSpec(memory_space=pltpu.MemorySpace.VMEM),
            pl.BlockSpec(memory_space=pl.ANY),
            pl.BlockSpec(memory_space=pl.ANY),
            pl.BlockSpec(memory_space=pltpu.MemorySpace.VMEM),
            pl.BlockSpec(memory_space=pltpu.MemorySpace.VMEM),
            pl.BlockSpec(memory_space=pltpu.MemorySpace.VMEM),
            pl.BlockSpec(memory_space=pltpu.MemorySpace.VMEM),
            pl.BlockSpec(memory_space=pltpu.MemorySpace.VMEM),
            pl.BlockSpec(memory_space=pltpu.MemorySpace.VMEM),
        ],
        out_specs=pl.BlockSpec(memory_space=pltpu.MemorySpace.VMEM),
        out_shape=jax.ShapeDtypeStruct((N, HID), jnp.float32),
        scratch_shapes=[
            pltpu.VMEM((F, V * N), jnp.bfloat16),
            pltpu.VMEM((F, V * N), jnp.bfloat16),
            pltpu.VMEM((2, V, N, BLOCK_D), jnp.int32),
            pltpu.VMEM((2, V, N, BLOCK_D), jnp.int32),
            pltpu.SemaphoreType.DMA((2, V)),
            pltpu.SemaphoreType.DMA((2, V)),
        ],
    )(inps, fw_adjs, bw_adjs, W_fw, b_fw, W_bw, b_bw, W1,
      b1.reshape(1, HID))
    return out


# speedup vs baseline: 1.0192x; 1.0113x over previous
import jax
import jax.numpy as jnp
from jax.experimental import pallas as pl
from jax.experimental.pallas import tpu as pltpu

N = 1024
HID = 128
V = 2
F = HID // 2
BLOCK_D = 256
GRID = N // BLOCK_D
NBUF = 3

_NORMAL = (((1,), (0,)), ((), ()))
_T_DIMNUMS = (((0,), (0,)), ((), ()))


def _bignn_kernel(x_ref, afw_ref, abw_ref, wfw_ref, bfw_ref, wbw_ref,
                  bbw_ref, w1_ref, b1_ref, out_ref,
                  hfw_ref, hbw_ref, buf_fw, buf_bw, sem_fw, sem_bw):
    x = x_ref[...]
    for w_ref, b_ref, h_ref in ((wfw_ref, bfw_ref, hfw_ref),
                                (wbw_ref, bbw_ref, hbw_ref)):
        for i in range(V):
            h = (jnp.dot(x, w_ref[i], preferred_element_type=jnp.float32)
                 + b_ref[i:i + 1, :])
            h_ref[:, pl.ds(i * N, N)] = jnp.swapaxes(
                h.astype(jnp.bfloat16), 0, 1)

    def _copies(j, slot):
        d = pl.ds(j * BLOCK_D, BLOCK_D)
        return [
            pltpu.make_async_copy(afw_ref.at[i, :, d], buf_fw.at[slot, i],
                                  sem_fw.at[slot, i])
            for i in range(V)
        ] + [
            pltpu.make_async_copy(abw_ref.at[i, :, d], buf_bw.at[slot, i],
                                  sem_bw.at[slot, i])
            for i in range(V)
        ]

    def _copy(j, slot):
        for c in _copies(j, slot):
            c.start()

    _copy(0, 0)
    _copy(1, 1)
    _copy(2, 2)
    for j in range(GRID):
        slot = j % NBUF
        for c in _copies(j, slot):
            c.wait()

        parts = []
        for buf, h_ref in ((buf_bw, hbw_ref), (buf_fw, hfw_ref)):
            acc = None
            for i in range(V):
                a = buf[slot, i].astype(jnp.bfloat16)
                agg_t = jax.lax.dot_general(
                    h_ref[:, pl.ds(i * N, N)], a, _NORMAL,
                    preferred_element_type=jnp.float32)
                r = jnp.maximum(agg_t, 0.0)
                acc = r if acc is None else acc + r
            parts.append(acc)
        summed_t = jnp.concatenate(parts, axis=0)

        if j + NBUF < GRID:
            _copy(j + NBUF, slot)

        d = pl.ds(j * BLOCK_D, BLOCK_D)
        feats = (jax.lax.dot_general(summed_t, w1_ref[...], _T_DIMNUMS,
                                     preferred_element_type=jnp.float32)
                 + b1_ref[...] + x_ref[d, :])
        out_ref[d, :] = feats


@jax.jit
def kernel(inps, fw_adjs, bw_adjs, W_fw, b_fw, W_bw, b_bw, W1, b1):
    out = pl.pallas_call(
        _bignn_kernel,
        in_specs=[
            pl.BlockSpec(memory_space=pltpu.MemorySpace.VMEM),
            pl.BlockSpec(memory_space=pl.ANY),
            pl.BlockSpec(memory_space=pl.ANY),
            pl.BlockSpec(memory_space=pltpu.MemorySpace.VMEM),
            pl.BlockSpec(memory_space=pltpu.MemorySpace.VMEM),
            pl.BlockSpec(memory_space=pltpu.MemorySpace.VMEM),
            pl.BlockSpec(memory_space=pltpu.MemorySpace.VMEM),
            pl.BlockSpec(memory_space=pltpu.MemorySpace.VMEM),
            pl.BlockSpec(memory_space=pltpu.MemorySpace.VMEM),
        ],
        out_specs=pl.BlockSpec(memory_space=pltpu.MemorySpace.VMEM),
        out_shape=jax.ShapeDtypeStruct((N, HID), jnp.float32),
        scratch_shapes=[
            pltpu.VMEM((F, V * N), jnp.bfloat16),
            pltpu.VMEM((F, V * N), jnp.bfloat16),
            pltpu.VMEM((NBUF, V, N, BLOCK_D), jnp.int32),
            pltpu.VMEM((NBUF, V, N, BLOCK_D), jnp.int32),
            pltpu.SemaphoreType.DMA((NBUF, V)),
            pltpu.SemaphoreType.DMA((NBUF, V)),
        ],
    )(inps, fw_adjs, bw_adjs, W_fw, b_fw, W_bw, b_bw, W1,
      b1.reshape(1, HID))
    return out
